# baseline (speedup 1.0000x reference)
import jax
import jax.numpy as jnp
from jax import lax
from jax.experimental import pallas as pl
from jax.experimental.pallas import tpu as pltpu
from jax.experimental.pallas import tpu_sc as plsc

B = 16384
D = 128
NC = 2
NS = 16
NW = NC * NS
BPW = B // NW
LANES = 16
NCH = 4
C = BPW // NCH


def _double(buf):
    def row_fn(r, carry):
        for j in range(D // LANES):
            sl = (r, pl.ds(j * LANES, LANES))
            v = buf[sl]
            buf[sl] = v + v
        return carry

    lax.fori_loop(0, C, row_fn, 0, unroll=2)


def _body(x_hbm, idx_hbm, out_hbm, idx_v, buf0, buf1, buf2, buf3, gsem, ssem):
    wid = lax.axis_index("s") * NC + lax.axis_index("c")
    base = wid * BPW
    bufs = (buf0, buf1, buf2, buf3)

    pltpu.sync_copy(idx_hbm.at[pl.ds(base, BPW)], idx_v)

    gathers = [
        pltpu.async_copy(x_hbm.at[idx_v.at[pl.ds(c * C, C)]], bufs[c], gsem)
        for c in range(NCH)
    ]
    scatters = []
    for c in range(NCH):
        gathers[c].wait()
        _double(bufs[c])
        scatters.append(
            pltpu.async_copy(bufs[c], out_hbm.at[pl.ds(base + c * C, C)], ssem)
        )
    for s in scatters:
        s.wait()


def kernel(input, indices):
    idx32 = indices.astype(jnp.int32)
    mesh = plsc.VectorSubcoreMesh(core_axis_name="c", subcore_axis_name="s")
    f = pl.kernel(
        _body,
        mesh=mesh,
        out_type=jax.ShapeDtypeStruct((B, D), jnp.float32),
        scratch_types=[
            pltpu.VMEM((BPW,), jnp.int32),
            pltpu.VMEM((C, D), jnp.float32),
            pltpu.VMEM((C, D), jnp.float32),
            pltpu.VMEM((C, D), jnp.float32),
            pltpu.VMEM((C, D), jnp.float32),
            pltpu.SemaphoreType.DMA,
            pltpu.SemaphoreType.DMA,
        ],
    )
    return f(input, idx32)


# speedup vs baseline: 1.0263x; 1.0263x over previous
import jax
import jax.numpy as jnp
from jax import lax
from jax.experimental import pallas as pl
from jax.experimental.pallas import tpu as pltpu
from jax.experimental.pallas import tpu_sc as plsc

B = 16384
D = 128
NC = 2
NS = 16
NW = NC * NS
BPW = B // NW
LANES = 16
NCH = 2
C = BPW // NCH


def _double(buf):
    def row_fn(r, carry):
        for j in range(D // LANES):
            sl = (r, pl.ds(j * LANES, LANES))
            v = buf[sl]
            buf[sl] = v + v
        return carry

    lax.fori_loop(0, C, row_fn, 0, unroll=2)


def _body(x_hbm, idx_hbm, out_hbm, idx0, idx1, buf0, buf1, isem, gsem, ssem):
    wid = lax.axis_index("s") * NC + lax.axis_index("c")
    base = wid * BPW

    i0 = pltpu.async_copy(idx_hbm.at[pl.ds(base, C)], idx0, isem)
    i1 = pltpu.async_copy(idx_hbm.at[pl.ds(base + C, C)], idx1, isem)
    i0.wait()
    g0 = pltpu.async_copy(x_hbm.at[idx0], buf0, gsem)
    i1.wait()
    g1 = pltpu.async_copy(x_hbm.at[idx1], buf1, gsem)
    g0.wait()
    _double(buf0)
    s0 = pltpu.async_copy(buf0, out_hbm.at[pl.ds(base, C)], ssem)
    g1.wait()
    _double(buf1)
    s1 = pltpu.async_copy(buf1, out_hbm.at[pl.ds(base + C, C)], ssem)
    s0.wait()
    s1.wait()


def kernel(input, indices):
    idx32 = indices.astype(jnp.int32)
    mesh = plsc.VectorSubcoreMesh(core_axis_name="c", subcore_axis_name="s")
    f = pl.kernel(
        _body,
        mesh=mesh,
        out_type=jax.ShapeDtypeStruct((B, D), jnp.float32),
        scratch_types=[
            pltpu.VMEM((C,), jnp.int32),
            pltpu.VMEM((C,), jnp.int32),
            pltpu.VMEM((C, D), jnp.float32),
            pltpu.VMEM((C, D), jnp.float32),
            pltpu.SemaphoreType.DMA,
            pltpu.SemaphoreType.DMA,
            pltpu.SemaphoreType.DMA,
        ],
    )
    return f(input, idx32)


# speedup vs baseline: 1.0284x; 1.0021x over previous
import jax
import jax.numpy as jnp
from jax import lax
from jax.experimental import pallas as pl
from jax.experimental.pallas import tpu as pltpu
from jax.experimental.pallas import tpu_sc as plsc

B = 16384
D = 128
NC = 2
NS = 16
NW = NC * NS
BPW = B // NW
LANES = 16
NCH = 2
C = BPW // NCH


def _double(buf):
    def row_fn(r, carry):
        for j in range(D // LANES):
            sl = (r, pl.ds(j * LANES, LANES))
            v = buf[sl]
            buf[sl] = v + v
        return carry

    lax.fori_loop(0, C, row_fn, 0, unroll=2)


def _body(x_hbm, idx_hbm, out_hbm, idx_v, buf0, buf1, gsem, ssem):
    wid = lax.axis_index("s") * NC + lax.axis_index("c")
    base = wid * BPW
    bufs = (buf0, buf1)

    pltpu.sync_copy(idx_hbm.at[pl.ds(base, BPW)], idx_v)

    g0 = pltpu.async_copy(x_hbm.at[idx_v.at[pl.ds(0, C)]], bufs[0], gsem)
    g1 = pltpu.async_copy(x_hbm.at[idx_v.at[pl.ds(C, C)]], bufs[1], gsem)
    g0.wait()
    _double(bufs[0])
    s0 = pltpu.async_copy(bufs[0], out_hbm.at[pl.ds(base, C)], ssem)
    g1.wait()
    _double(bufs[1])
    s1 = pltpu.async_copy(bufs[1], out_hbm.at[pl.ds(base + C, C)], ssem)
    s0.wait()
    s1.wait()


def kernel(input, indices):
    idx32 = indices.astype(jnp.int32)
    mesh = plsc.VectorSubcoreMesh(core_axis_name="c", subcore_axis_name="s")
    f = pl.kernel(
        _body,
        mesh=mesh,
        out_type=jax.ShapeDtypeStruct((B, D), jnp.float32),
        scratch_types=[
            pltpu.VMEM((BPW,), jnp.int32),
            pltpu.VMEM((C, D), jnp.float32),
            pltpu.VMEM((C, D), jnp.float32),
            pltpu.SemaphoreType.DMA,
            pltpu.SemaphoreType.DMA,
        ],
    )
    return f(input, idx32)
